# baseline (speedup 1.0000x reference)
import functools

import jax
import jax.numpy as jnp
from jax import lax
from jax.experimental import pallas as pl
from jax.experimental.pallas import tpu as pltpu
from jax.experimental.pallas import tpu_sc as plsc

N_NODES = 10000
NPAD = 10240
D = 128
NC = 2
NS = 16
NW = NC * NS
K = 64
NBUF = 3
ROWBLK = 1024
F32 = jnp.float32
I32 = jnp.int32


def _sc_mesh():
    return plsc.VectorSubcoreMesh(core_axis_name="c", subcore_axis_name="s")


def _deg_body(epw, col_hbm, ew_hbm, degp_hbm, colb, ewb, degv):
    cid = lax.axis_index("c")
    sid = lax.axis_index("s")
    wid = sid * NC + cid

    def zero(i, c):
        degv[pl.ds(i * 16, 16)] = jnp.zeros((16,), F32)
        return c

    lax.fori_loop(0, NPAD // 16, zero, 0)
    pltpu.sync_copy(col_hbm.at[pl.ds(wid * epw, epw)], colb)
    pltpu.sync_copy(ew_hbm.at[pl.ds(wid * epw, epw)], ewb)

    def edge(i, c):
        b = i * 16
        plsc.addupdate_scatter(degv, [colb[pl.ds(b, 16)]], ewb[pl.ds(b, 16)])
        return c

    lax.fori_loop(0, epw // 16, edge, 0)
    pltpu.sync_copy(degv, degp_hbm.at[wid])


def _sc_degree(col_p, ew_p, epw):
    kfn = functools.partial(
        pl.kernel,
        out_type=jax.ShapeDtypeStruct((NW, NPAD), F32),
        mesh=_sc_mesh(),
        compiler_params=pltpu.CompilerParams(needs_layout_passes=False),
        scratch_types=[
            pltpu.VMEM((epw,), I32),
            pltpu.VMEM((epw,), F32),
            pltpu.VMEM((NPAD,), F32),
        ],
    )(functools.partial(_deg_body, epw))
    return kfn(col_p, ew_p)


def _msg_body(nchunks, unroll, zs_hbm, row_hbm, colew_hbm, out_hbm,
              acc_sh, rowb, rows0, rows1, rows2,
              cb0, cb1, cb2,
              sg0, sg1, sg2, ss0, ss1, ss2, sc0, sc1, sc2):
    cid = lax.axis_index("c")
    sid = lax.axis_index("s")
    wid = sid * NC + cid
    rows_per_tile = NPAD // NS
    rows = [rows0, rows1, rows2]
    combo = [cb0, cb1, cb2]
    sg = [sg0, sg1, sg2]
    ss = [ss0, ss1, ss2]
    sc = [sc0, sc1, sc2]

    pltpu.sync_copy(row_hbm.at[wid], rowb)

    @plsc.parallel_loop(0, K)
    def _(i):
        for g in range(D // 16):
            rows0[i, pl.ds(g * 16, 16)] = jnp.zeros((16,), F32)

    def zacc(j, c):
        pltpu.sync_copy(rows0, acc_sh.at[pl.ds(sid * rows_per_tile + j * K, K)])
        return c

    lax.fori_loop(0, rows_per_tile // K, zacc, 0)

    for b in range(NBUF - 1):
        pltpu.async_copy(colew_hbm.at[wid, b], combo[b], sc[b])
        pltpu.async_copy(zs_hbm.at[rowb.at[b]], rows[b], sg[b])
    plsc.subcore_barrier()

    def scale_chunk(rbuf, ew_i32_row):
        @plsc.parallel_loop(0, K, unroll=unroll)
        def _(k):
            spl = plsc.bitcast(
                plsc.load_gather(ew_i32_row, [jnp.full((16,), k, I32)]), F32)
            for g in range(D // 16):
                seg = rbuf[k, pl.ds(g * 16, 16)]
                rbuf[k, pl.ds(g * 16, 16)] = seg * spl

    def outer(jo, c):
        for b in range(NBUF):
            j = jo * NBUF + b
            bg = (b + NBUF - 1) % NBUF
            pltpu.make_async_copy(colew_hbm.at[wid, j], combo[b], sc[b]).wait()
            pltpu.make_async_copy(zs_hbm.at[rowb.at[j]], rows[b], sg[b]).wait()
            scale_chunk(rows[b], combo[b].at[1])
            pltpu.async_copy(rows[b], acc_sh.at[combo[b].at[0]], ss[b],
                             add=True)

            @pl.when(j >= 1)
            def _():
                pltpu.make_async_copy(
                    rows[bg], acc_sh.at[combo[bg].at[0]], ss[bg]).wait()

            @pl.when(j + NBUF - 1 < nchunks)
            def _():
                pltpu.async_copy(
                    colew_hbm.at[wid, j + NBUF - 1], combo[bg], sc[bg])
                pltpu.async_copy(
                    zs_hbm.at[rowb.at[j + NBUF - 1]], rows[bg], sg[bg])
        return c

    lax.fori_loop(0, nchunks // NBUF, outer, 0)

    lastb = (nchunks - 1) % NBUF
    pltpu.make_async_copy(
        rows[lastb], acc_sh.at[combo[lastb].at[0]], ss[lastb]).wait()
    plsc.subcore_barrier()

    def wout(j, c):
        r = sid * rows_per_tile + j * K
        pltpu.sync_copy(acc_sh.at[pl.ds(r, K)], rows0)
        pltpu.sync_copy(rows0, out_hbm.at[cid, pl.ds(r, K)])
        return c

    lax.fori_loop(0, rows_per_tile // K, wout, 0)


def _sc_message(zs, row3, colew, nchunks, unroll=4):
    kfn = functools.partial(
        pl.kernel,
        out_type=jax.ShapeDtypeStruct((NC, NPAD, D), F32),
        mesh=_sc_mesh(),
        compiler_params=pltpu.CompilerParams(needs_layout_passes=False),
        scratch_types=[
            pltpu.VMEM_SHARED((NPAD, D), F32),
            pltpu.VMEM((nchunks, K), I32),
        ] + [pltpu.VMEM((K, D), F32)] * NBUF
          + [pltpu.VMEM((2, K), I32)] * NBUF
          + [pltpu.SemaphoreType.DMA] * (3 * NBUF),
    )(functools.partial(_msg_body, nchunks, unroll))
    return kfn(zs, row3, colew)


def _mm_body(x_ref, w_ref, o_ref):
    o_ref[...] = jnp.dot(x_ref[...], w_ref[...], preferred_element_type=F32)


def _tc_matmul(x, w):
    return pl.pallas_call(
        _mm_body,
        grid=(NPAD // ROWBLK,),
        in_specs=[
            pl.BlockSpec((ROWBLK, D), lambda i: (i, 0)),
            pl.BlockSpec((D, D), lambda i: (0, 0)),
        ],
        out_specs=pl.BlockSpec((ROWBLK, D), lambda i: (i, 0)),
        out_shape=jax.ShapeDtypeStruct((NPAD, D), F32),
    )(x, w)


def _scale1_body(z_ref, deg_ref, zs_ref, dis_ref):
    dis = lax.rsqrt(deg_ref[...] + 1.0)
    dis_ref[...] = dis
    zs_ref[...] = z_ref[...] * dis


def _tc_scale1(z, deg_col):
    return pl.pallas_call(
        _scale1_body,
        grid=(NPAD // ROWBLK,),
        in_specs=[
            pl.BlockSpec((ROWBLK, D), lambda i: (i, 0)),
            pl.BlockSpec((ROWBLK, 1), lambda i: (i, 0)),
        ],
        out_specs=[
            pl.BlockSpec((ROWBLK, D), lambda i: (i, 0)),
            pl.BlockSpec((ROWBLK, 1), lambda i: (i, 0)),
        ],
        out_shape=[
            jax.ShapeDtypeStruct((NPAD, D), F32),
            jax.ShapeDtypeStruct((NPAD, 1), F32),
        ],
    )(z, deg_col)


def _mid_body(acc_ref, zs_ref, dis_ref, w_ref, b_ref, o_ref):
    dis = dis_ref[...]
    pre = (acc_ref[0] + acc_ref[1] + zs_ref[...]) * dis + b_ref[...]
    h = jnp.maximum(pre, 0.0)
    o_ref[...] = jnp.dot(h, w_ref[...], preferred_element_type=F32) * dis


def _tc_mid(acc, zs, dis_col, w2, b1):
    return pl.pallas_call(
        _mid_body,
        grid=(NPAD // ROWBLK,),
        in_specs=[
            pl.BlockSpec((NC, ROWBLK, D), lambda i: (0, i, 0)),
            pl.BlockSpec((ROWBLK, D), lambda i: (i, 0)),
            pl.BlockSpec((ROWBLK, 1), lambda i: (i, 0)),
            pl.BlockSpec((D, D), lambda i: (0, 0)),
            pl.BlockSpec((1, D), lambda i: (0, 0)),
        ],
        out_specs=pl.BlockSpec((ROWBLK, D), lambda i: (i, 0)),
        out_shape=jax.ShapeDtypeStruct((NPAD, D), F32),
    )(acc, zs, dis_col, w2, b1)


def _final_body(acc_ref, zs_ref, dis_ref, b_ref, o_ref):
    o_ref[...] = ((acc_ref[0] + acc_ref[1] + zs_ref[...]) * dis_ref[...]
                  + b_ref[...])


def _tc_final(acc, zs, dis_col, b2):
    return pl.pallas_call(
        _final_body,
        grid=(NPAD // ROWBLK,),
        in_specs=[
            pl.BlockSpec((NC, ROWBLK, D), lambda i: (0, i, 0)),
            pl.BlockSpec((ROWBLK, D), lambda i: (i, 0)),
            pl.BlockSpec((ROWBLK, 1), lambda i: (i, 0)),
            pl.BlockSpec((1, D), lambda i: (0, 0)),
        ],
        out_specs=pl.BlockSpec((ROWBLK, D), lambda i: (i, 0)),
        out_shape=jax.ShapeDtypeStruct((NPAD, D), F32),
    )(acc, zs, dis_col, b2)


def kernel(x, edge_index, weight, W1, b1, W2, b2):
    e = weight.shape[0]
    row = edge_index[0].astype(I32)
    col = edge_index[1].astype(I32)
    ew = weight.astype(F32)

    epw_d = ((e + NW * 16 - 1) // (NW * 16)) * 16
    e_pad_d = epw_d * NW
    col_d = jnp.pad(col, (0, e_pad_d - e))
    ew_d = jnp.pad(ew, (0, e_pad_d - e))

    nchunks = ((e + NW * K - 1) // (NW * K) + NBUF - 1) // NBUF * NBUF
    e_pad_m = NW * nchunks * K
    row3 = jnp.pad(row, (0, e_pad_m - e)).reshape(NW, nchunks, K)
    col4 = jnp.pad(col, (0, e_pad_m - e)).reshape(NW, nchunks, 1, K)
    ew4 = jnp.pad(ew, (0, e_pad_m - e)).reshape(NW, nchunks, 1, K)
    colew = jnp.concatenate(
        [col4, jax.lax.bitcast_convert_type(ew4, I32)], axis=2)
    x_pad = jnp.pad(x, ((0, NPAD - x.shape[0]), (0, 0)))

    z1 = _tc_matmul(x_pad, W1)
    degp = _sc_degree(col_d, ew_d, epw_d)
    deg_col = jnp.sum(degp, axis=0).reshape(NPAD, 1)
    zs1, dis_col = _tc_scale1(z1, deg_col)
    acc1 = _sc_message(zs1, row3, colew, nchunks)
    zs2 = _tc_mid(acc1, zs1, dis_col, W2, b1.reshape(1, D))
    acc2 = _sc_message(zs2, row3, colew, nchunks)
    out = _tc_final(acc2, zs2, dis_col, b2.reshape(1, D))
    return out[:N_NODES]


# speedup vs baseline: 1.6516x; 1.6516x over previous
import functools

import jax
import jax.numpy as jnp
from jax import lax
from jax.experimental import pallas as pl
from jax.experimental.pallas import tpu as pltpu
from jax.experimental.pallas import tpu_sc as plsc

N_NODES = 10000
NPAD = 10240
D = 128
NC = 2
NS = 16
NW = NC * NS
K = 128
NBUF = 3
ROWBLK = 1024
F32 = jnp.float32
I32 = jnp.int32


def _sc_mesh():
    return plsc.VectorSubcoreMesh(core_axis_name="c", subcore_axis_name="s")


def _deg_body(epw, col_hbm, ew_hbm, degp_hbm, colb, ewb, degv):
    cid = lax.axis_index("c")
    sid = lax.axis_index("s")
    wid = sid * NC + cid

    def zero(i, c):
        degv[pl.ds(i * 16, 16)] = jnp.zeros((16,), F32)
        return c

    lax.fori_loop(0, NPAD // 16, zero, 0)
    pltpu.sync_copy(col_hbm.at[pl.ds(wid * epw, epw)], colb)
    pltpu.sync_copy(ew_hbm.at[pl.ds(wid * epw, epw)], ewb)

    def edge(i, c):
        b = i * 16
        plsc.addupdate_scatter(degv, [colb[pl.ds(b, 16)]], ewb[pl.ds(b, 16)])
        return c

    lax.fori_loop(0, epw // 16, edge, 0)
    pltpu.sync_copy(degv, degp_hbm.at[wid])


def _sc_degree(col_p, ew_p, epw):
    kfn = functools.partial(
        pl.kernel,
        out_type=jax.ShapeDtypeStruct((NW, NPAD), F32),
        mesh=_sc_mesh(),
        compiler_params=pltpu.CompilerParams(needs_layout_passes=False),
        scratch_types=[
            pltpu.VMEM((epw,), I32),
            pltpu.VMEM((epw,), F32),
            pltpu.VMEM((NPAD,), F32),
        ],
    )(functools.partial(_deg_body, epw))
    return kfn(col_p, ew_p)


DH = D // NC
DQ = DH // 2


def _msg_body(nchunks, unroll, zsh_hbm, rc_hbm, ew_hbm, out_hbm,
              acc_sh, rcb, ewb, gb0, gb1, gb2, sb0, sb1, sb2,
              rv0, rv1, rv2, cv0, cv1, cv2,
              sg0, sg1, sg2, ss0, ss1, ss2):
    cid = lax.axis_index("c")
    sid = lax.axis_index("s")
    rows_per_tile = NPAD // NS
    gb = [gb0, gb1, gb2]
    sb = [sb0, sb1, sb2]
    rv = [rv0, rv1, rv2]
    cv = [cv0, cv1, cv2]
    sg = [sg0, sg1, sg2]
    ss = [ss0, ss1, ss2]
    zs_half = zsh_hbm.at[cid]
    himask = jnp.full((16,), -65536, I32)
    lomask = jnp.full((16,), 65535, I32)

    pltpu.sync_copy(rc_hbm.at[sid], rcb)
    pltpu.sync_copy(ew_hbm.at[sid], ewb)

    def mat_rows(j, dst):
        rc_row = rcb.at[j]

        @plsc.parallel_loop(0, K // 16)
        def _(g):
            dst[pl.ds(g * 16, 16)] = rc_row[pl.ds(g * 16, 16)] & lomask

    def mat_cols(j, dst):
        rc_row = rcb.at[j]

        @plsc.parallel_loop(0, K // 16)
        def _(g):
            dst[pl.ds(g * 16, 16)] = lax.shift_right_logical(
                rc_row[pl.ds(g * 16, 16)], 16)

    @plsc.parallel_loop(0, K)
    def _(i):
        for g in range(DH // 16):
            sb0[i, pl.ds(g * 16, 16)] = jnp.zeros((16,), F32)

    def zacc(j, c):
        pltpu.sync_copy(sb0, acc_sh.at[pl.ds(sid * rows_per_tile + j * K, K)])
        return c

    lax.fori_loop(0, rows_per_tile // K, zacc, 0)

    for b in range(NBUF - 1):
        mat_rows(b, rv[b])
        pltpu.async_copy(zs_half.at[rv[b]], gb[b], sg[b])
    plsc.subcore_barrier()

    def scale_chunk(gbuf, sbuf, j):
        ew_row = ewb.at[j]

        @plsc.parallel_loop(0, K, unroll=unroll)
        def _(k):
            spl = plsc.load_gather(ew_row, [jnp.full((16,), k, I32)])
            for g in range(DQ // 16):
                xi = gbuf[k, pl.ds(g * 16, 16)]
                lo = plsc.bitcast(lax.shift_left(xi, 16), F32)
                hi = plsc.bitcast(xi & himask, F32)
                sbuf[k, pl.ds(g * 16, 16)] = lo * spl
                sbuf[k, pl.ds(DQ + g * 16, 16)] = hi * spl

    def outer(jo, c):
        for b in range(NBUF):
            j = jo * NBUF + b
            bg = (b + NBUF - 1) % NBUF
            pltpu.make_async_copy(zs_half.at[rv[b]], gb[b], sg[b]).wait()
            scale_chunk(gb[b], sb[b], j)
            mat_cols(j, cv[b])
            pltpu.async_copy(sb[b], acc_sh.at[cv[b]], ss[b], add=True)

            @pl.when(j >= 1)
            def _():
                pltpu.make_async_copy(sb[bg], acc_sh.at[cv[bg]], ss[bg]).wait()

            @pl.when(j + NBUF - 1 < nchunks)
            def _():
                mat_rows(j + NBUF - 1, rv[bg])
                pltpu.async_copy(zs_half.at[rv[bg]], gb[bg], sg[bg])
        return c

    lax.fori_loop(0, nchunks // NBUF, outer, 0)

    lastb = (nchunks - 1) % NBUF
    pltpu.make_async_copy(sb[lastb], acc_sh.at[cv[lastb]], ss[lastb]).wait()
    plsc.subcore_barrier()

    def wout(j, c):
        r = sid * rows_per_tile + j * K
        pltpu.sync_copy(acc_sh.at[pl.ds(r, K)], sb0)
        pltpu.sync_copy(sb0, out_hbm.at[cid, pl.ds(r, K)])
        return c

    lax.fori_loop(0, rows_per_tile // K, wout, 0)


def _sc_message(zsh, rc3, ew3, nchunks, unroll=4):
    kfn = functools.partial(
        pl.kernel,
        out_type=jax.ShapeDtypeStruct((NC, NPAD, DH), F32),
        mesh=_sc_mesh(),
        compiler_params=pltpu.CompilerParams(
            needs_layout_passes=False, use_tc_tiling_on_sc=False),
        scratch_types=[
            pltpu.VMEM_SHARED((NPAD, DH), F32),
            pltpu.VMEM((nchunks, K), I32),
            pltpu.VMEM((nchunks, K), F32),
        ] + [pltpu.VMEM((K, DQ), I32)] * NBUF
          + [pltpu.VMEM((K, DH), F32)] * NBUF
          + [pltpu.VMEM((K,), I32)] * (2 * NBUF)
          + [pltpu.SemaphoreType.DMA] * (2 * NBUF),
    )(functools.partial(_msg_body, nchunks, unroll))
    return kfn(zsh, rc3, ew3)


def _mm_body(x_ref, w_ref, o_ref):
    o_ref[...] = jnp.dot(x_ref[...], w_ref[...], preferred_element_type=F32)


def _tc_matmul(x, w):
    return pl.pallas_call(
        _mm_body,
        grid=(NPAD // ROWBLK,),
        in_specs=[
            pl.BlockSpec((ROWBLK, D), lambda i: (i, 0)),
            pl.BlockSpec((D, D), lambda i: (0, 0)),
        ],
        out_specs=pl.BlockSpec((ROWBLK, D), lambda i: (i, 0)),
        out_shape=jax.ShapeDtypeStruct((NPAD, D), F32),
    )(x, w)


def _scale1_body(z_ref, deg_ref, zs_ref, dis_ref):
    dis = lax.rsqrt(deg_ref[...] + 1.0)
    dis_ref[...] = dis
    zs_ref[...] = z_ref[...] * dis


def _tc_scale1(z, deg_col):
    return pl.pallas_call(
        _scale1_body,
        grid=(NPAD // ROWBLK,),
        in_specs=[
            pl.BlockSpec((ROWBLK, D), lambda i: (i, 0)),
            pl.BlockSpec((ROWBLK, 1), lambda i: (i, 0)),
        ],
        out_specs=[
            pl.BlockSpec((ROWBLK, D), lambda i: (i, 0)),
            pl.BlockSpec((ROWBLK, 1), lambda i: (i, 0)),
        ],
        out_shape=[
            jax.ShapeDtypeStruct((NPAD, D), F32),
            jax.ShapeDtypeStruct((NPAD, 1), F32),
        ],
    )(z, deg_col)


def _mid_body(acc_ref, zs_ref, dis_ref, w_ref, b_ref, o_ref):
    dis = dis_ref[...]
    full = jnp.concatenate([acc_ref[0], acc_ref[1]], axis=1) + zs_ref[...]
    pre = full * dis + b_ref[...]
    h = jnp.maximum(pre, 0.0)
    o_ref[...] = jnp.dot(h, w_ref[...], preferred_element_type=F32) * dis


def _tc_mid(acc, zs, dis_col, w2, b1):
    return pl.pallas_call(
        _mid_body,
        grid=(NPAD // ROWBLK,),
        in_specs=[
            pl.BlockSpec((NC, ROWBLK, DH), lambda i: (0, i, 0)),
            pl.BlockSpec((ROWBLK, D), lambda i: (i, 0)),
            pl.BlockSpec((ROWBLK, 1), lambda i: (i, 0)),
            pl.BlockSpec((D, D), lambda i: (0, 0)),
            pl.BlockSpec((1, D), lambda i: (0, 0)),
        ],
        out_specs=pl.BlockSpec((ROWBLK, D), lambda i: (i, 0)),
        out_shape=jax.ShapeDtypeStruct((NPAD, D), F32),
    )(acc, zs, dis_col, w2, b1)


def _final_body(acc_ref, zs_ref, dis_ref, b_ref, o_ref):
    full = jnp.concatenate([acc_ref[0], acc_ref[1]], axis=1) + zs_ref[...]
    o_ref[...] = full * dis_ref[...] + b_ref[...]


def _tc_final(acc, zs, dis_col, b2):
    return pl.pallas_call(
        _final_body,
        grid=(NPAD // ROWBLK,),
        in_specs=[
            pl.BlockSpec((NC, ROWBLK, DH), lambda i: (0, i, 0)),
            pl.BlockSpec((ROWBLK, D), lambda i: (i, 0)),
            pl.BlockSpec((ROWBLK, 1), lambda i: (i, 0)),
            pl.BlockSpec((1, D), lambda i: (0, 0)),
        ],
        out_specs=pl.BlockSpec((ROWBLK, D), lambda i: (i, 0)),
        out_shape=jax.ShapeDtypeStruct((NPAD, D), F32),
    )(acc, zs, dis_col, b2)


def kernel(x, edge_index, weight, W1, b1, W2, b2):
    e = weight.shape[0]
    row = edge_index[0].astype(I32)
    col = edge_index[1].astype(I32)
    ew = weight.astype(F32)

    epw_d = ((e + NW * 16 - 1) // (NW * 16)) * 16
    e_pad_d = epw_d * NW
    col_d = jnp.pad(col, (0, e_pad_d - e))
    ew_d = jnp.pad(ew, (0, e_pad_d - e))

    nchunks = ((e + NS * K - 1) // (NS * K) + NBUF - 1) // NBUF * NBUF
    e_pad_m = NS * nchunks * K
    row_m = jnp.pad(row, (0, e_pad_m - e))
    col_m = jnp.pad(col, (0, e_pad_m - e))
    rc3 = (row_m | (col_m << 16)).reshape(NS, nchunks, K)
    ew3 = jnp.pad(ew, (0, e_pad_m - e)).reshape(NS, nchunks, K)
    x_pad = jnp.pad(x, ((0, NPAD - x.shape[0]), (0, 0)))

    def pack_bf16(zs):
        zsb = zs.astype(jnp.bfloat16).reshape(NPAD, NC, 2, DQ)
        arr = zsb.transpose(1, 0, 3, 2)
        return jax.lax.bitcast_convert_type(arr, I32)

    z1 = _tc_matmul(x_pad, W1)
    degp = _sc_degree(col_d, ew_d, epw_d)
    deg_col = jnp.sum(degp, axis=0).reshape(NPAD, 1)
    zs1, dis_col = _tc_scale1(z1, deg_col)
    acc1 = _sc_message(pack_bf16(zs1), rc3, ew3, nchunks)
    zs2 = _tc_mid(acc1, zs1, dis_col, W2, b1.reshape(1, D))
    acc2 = _sc_message(pack_bf16(zs2), rc3, ew3, nchunks)
    out = _tc_final(acc2, zs2, dis_col, b2.reshape(1, D))
    return out[:N_NODES]


# speedup vs baseline: 1.7644x; 1.0683x over previous
import functools

import jax
import jax.numpy as jnp
from jax import lax
from jax.experimental import pallas as pl
from jax.experimental.pallas import tpu as pltpu
from jax.experimental.pallas import tpu_sc as plsc

N_NODES = 10000
NPAD = 10240
D = 128
NC = 2
NS = 16
NW = NC * NS
K = 128
NBUF = 3
ROWBLK = 1000
F32 = jnp.float32
I32 = jnp.int32


def _sc_mesh():
    return plsc.VectorSubcoreMesh(core_axis_name="c", subcore_axis_name="s")


def _deg_body(epw, col_hbm, ew_hbm, degp_hbm, colb, ewb, degv):
    cid = lax.axis_index("c")
    sid = lax.axis_index("s")
    wid = sid * NC + cid

    def zero(i, c):
        degv[pl.ds(i * 16, 16)] = jnp.zeros((16,), F32)
        return c

    lax.fori_loop(0, NPAD // 16, zero, 0)
    pltpu.sync_copy(col_hbm.at[pl.ds(wid * epw, epw)], colb)
    pltpu.sync_copy(ew_hbm.at[pl.ds(wid * epw, epw)], ewb)

    def edge(i, c):
        b = i * 16
        plsc.addupdate_scatter(degv, [colb[pl.ds(b, 16)]], ewb[pl.ds(b, 16)])
        return c

    lax.fori_loop(0, epw // 16, edge, 0)
    pltpu.sync_copy(degv, degp_hbm.at[wid])


def _sc_degree(col_p, ew_p, epw):
    kfn = functools.partial(
        pl.kernel,
        out_type=jax.ShapeDtypeStruct((NW, NPAD), F32),
        mesh=_sc_mesh(),
        compiler_params=pltpu.CompilerParams(needs_layout_passes=False),
        scratch_types=[
            pltpu.VMEM((epw,), I32),
            pltpu.VMEM((epw,), F32),
            pltpu.VMEM((NPAD,), F32),
        ],
    )(functools.partial(_deg_body, epw))
    return kfn(col_p, ew_p)


DH = D // NC
DQ = DH // 2


def _msg_body(nchunks, unroll, zsh_hbm, rc_hbm, ew_hbm, out_hbm,
              acc_sh, rcb, ewb, gb0, gb1, gb2, sb0, sb1, sb2,
              rv0, rv1, rv2, cv0, cv1, cv2,
              sg0, sg1, sg2, ss0, ss1, ss2):
    cid = lax.axis_index("c")
    sid = lax.axis_index("s")
    rows_per_tile = NPAD // NS
    gb = [gb0, gb1, gb2]
    sb = [sb0, sb1, sb2]
    rv = [rv0, rv1, rv2]
    cv = [cv0, cv1, cv2]
    sg = [sg0, sg1, sg2]
    ss = [ss0, ss1, ss2]
    zs_half = zsh_hbm.at[cid]
    himask = jnp.full((16,), -65536, I32)
    lomask = jnp.full((16,), 65535, I32)

    pltpu.sync_copy(rc_hbm.at[sid], rcb)
    pltpu.sync_copy(ew_hbm.at[sid], ewb)

    def mat_rows(j, dst):
        rc_row = rcb.at[j]

        @plsc.parallel_loop(0, K // 16)
        def _(g):
            dst[pl.ds(g * 16, 16)] = rc_row[pl.ds(g * 16, 16)] & lomask

    def mat_cols(j, dst):
        rc_row = rcb.at[j]

        @plsc.parallel_loop(0, K // 16)
        def _(g):
            dst[pl.ds(g * 16, 16)] = lax.shift_right_logical(
                rc_row[pl.ds(g * 16, 16)], 16)

    @plsc.parallel_loop(0, K)
    def _(i):
        for g in range(DH // 16):
            sb0[i, pl.ds(g * 16, 16)] = jnp.zeros((16,), F32)

    def zacc(j, c):
        pltpu.sync_copy(sb0, acc_sh.at[pl.ds(sid * rows_per_tile + j * K, K)])
        return c

    lax.fori_loop(0, rows_per_tile // K, zacc, 0)

    for b in range(NBUF - 1):
        mat_rows(b, rv[b])
        pltpu.async_copy(zs_half.at[rv[b]], gb[b], sg[b])
    plsc.subcore_barrier()

    def scale_chunk(gbuf, sbuf, j):
        ew_row = ewb.at[j]

        @plsc.parallel_loop(0, K, unroll=unroll)
        def _(k):
            spl = plsc.load_gather(ew_row, [jnp.full((16,), k, I32)])
            for g in range(DQ // 16):
                xi = gbuf[k, pl.ds(g * 16, 16)]
                lo = plsc.bitcast(lax.shift_left(xi, 16), F32)
                hi = plsc.bitcast(xi & himask, F32)
                sbuf[k, pl.ds(g * 16, 16)] = lo * spl
                sbuf[k, pl.ds(DQ + g * 16, 16)] = hi * spl

    def outer(jo, c):
        for b in range(NBUF):
            j = jo * NBUF + b
            bg = (b + NBUF - 1) % NBUF
            pltpu.make_async_copy(zs_half.at[rv[b]], gb[b], sg[b]).wait()
            scale_chunk(gb[b], sb[b], j)
            mat_cols(j, cv[b])
            pltpu.async_copy(sb[b], acc_sh.at[cv[b]], ss[b], add=True)

            @pl.when(j >= 1)
            def _():
                pltpu.make_async_copy(sb[bg], acc_sh.at[cv[bg]], ss[bg]).wait()

            @pl.when(j + NBUF - 1 < nchunks)
            def _():
                mat_rows(j + NBUF - 1, rv[bg])
                pltpu.async_copy(zs_half.at[rv[bg]], gb[bg], sg[bg])
        return c

    lax.fori_loop(0, nchunks // NBUF, outer, 0)

    lastb = (nchunks - 1) % NBUF
    pltpu.make_async_copy(sb[lastb], acc_sh.at[cv[lastb]], ss[lastb]).wait()
    plsc.subcore_barrier()

    def wout(j, c):
        r = sid * rows_per_tile + j * K
        pltpu.sync_copy(acc_sh.at[pl.ds(r, K)], sb0)
        pltpu.sync_copy(sb0, out_hbm.at[cid, pl.ds(r, K)])
        return c

    lax.fori_loop(0, rows_per_tile // K, wout, 0)


def _sc_message(zsh, rc3, ew3, nchunks, unroll=4):
    kfn = functools.partial(
        pl.kernel,
        out_type=jax.ShapeDtypeStruct((NC, NPAD, DH), F32),
        mesh=_sc_mesh(),
        compiler_params=pltpu.CompilerParams(
            needs_layout_passes=False, use_tc_tiling_on_sc=False),
        scratch_types=[
            pltpu.VMEM_SHARED((NPAD, DH), F32),
            pltpu.VMEM((nchunks, K), I32),
            pltpu.VMEM((nchunks, K), F32),
        ] + [pltpu.VMEM((K, DQ), I32)] * NBUF
          + [pltpu.VMEM((K, DH), F32)] * NBUF
          + [pltpu.VMEM((K,), I32)] * (2 * NBUF)
          + [pltpu.SemaphoreType.DMA] * (2 * NBUF),
    )(functools.partial(_msg_body, nchunks, unroll))
    return kfn(zsh, rc3, ew3)


def _mm_body(x_ref, w_ref, o_ref):
    o_ref[...] = jnp.dot(x_ref[...], w_ref[...], preferred_element_type=F32)


def _tc_matmul(x, w):
    return pl.pallas_call(
        _mm_body,
        grid=(N_NODES // ROWBLK,),
        in_specs=[
            pl.BlockSpec((ROWBLK, D), lambda i: (i, 0)),
            pl.BlockSpec((D, D), lambda i: (0, 0)),
        ],
        out_specs=pl.BlockSpec((ROWBLK, D), lambda i: (i, 0)),
        out_shape=jax.ShapeDtypeStruct((N_NODES, D), F32),
    )(x, w)


def _pack_halves(zs, oh_ref):
    for h in range(NC):
        a = zs[:, h * DH: h * DH + DQ]
        b = zs[:, h * DH + DQ: h * DH + 2 * DQ]
        ai = lax.bitcast_convert_type(a.astype(jnp.bfloat16).astype(F32), I32)
        bi = lax.bitcast_convert_type(b.astype(jnp.bfloat16).astype(F32), I32)
        oh_ref[h] = lax.shift_right_logical(ai, 16) | (bi & (-65536))


def _scale1_body(z_ref, deg_ref, zs_ref, zsh_ref, dis_ref):
    dis = lax.rsqrt(deg_ref[...] + 1.0)
    dis_ref[...] = dis
    zs = z_ref[...] * dis
    zs_ref[...] = zs
    _pack_halves(zs, zsh_ref)


def _tc_scale1(z, deg_col):
    return pl.pallas_call(
        _scale1_body,
        grid=(N_NODES // ROWBLK,),
        in_specs=[
            pl.BlockSpec((ROWBLK, D), lambda i: (i, 0)),
            pl.BlockSpec((ROWBLK, 1), lambda i: (i, 0)),
        ],
        out_specs=[
            pl.BlockSpec((ROWBLK, D), lambda i: (i, 0)),
            pl.BlockSpec((NC, ROWBLK, DQ), lambda i: (0, i, 0)),
            pl.BlockSpec((ROWBLK, 1), lambda i: (i, 0)),
        ],
        out_shape=[
            jax.ShapeDtypeStruct((N_NODES, D), F32),
            jax.ShapeDtypeStruct((NC, N_NODES, DQ), I32),
            jax.ShapeDtypeStruct((N_NODES, 1), F32),
        ],
    )(z, deg_col)


def _mid_body(acc_ref, zs_ref, dis_ref, w_ref, b_ref, o_ref, oh_ref):
    dis = dis_ref[...]
    full = jnp.concatenate([acc_ref[0], acc_ref[1]], axis=1) + zs_ref[...]
    pre = full * dis + b_ref[...]
    h = jnp.maximum(pre, 0.0)
    z2 = jnp.dot(h, w_ref[...], preferred_element_type=F32) * dis
    o_ref[...] = z2
    _pack_halves(z2, oh_ref)


def _tc_mid(acc, zs, dis_col, w2, b1):
    return pl.pallas_call(
        _mid_body,
        grid=(N_NODES // ROWBLK,),
        in_specs=[
            pl.BlockSpec((NC, ROWBLK, DH), lambda i: (0, i, 0)),
            pl.BlockSpec((ROWBLK, D), lambda i: (i, 0)),
            pl.BlockSpec((ROWBLK, 1), lambda i: (i, 0)),
            pl.BlockSpec((D, D), lambda i: (0, 0)),
            pl.BlockSpec((1, D), lambda i: (0, 0)),
        ],
        out_specs=[
            pl.BlockSpec((ROWBLK, D), lambda i: (i, 0)),
            pl.BlockSpec((NC, ROWBLK, DQ), lambda i: (0, i, 0)),
        ],
        out_shape=[
            jax.ShapeDtypeStruct((N_NODES, D), F32),
            jax.ShapeDtypeStruct((NC, N_NODES, DQ), I32),
        ],
    )(acc, zs, dis_col, w2, b1)


def _final_body(acc_ref, zs_ref, dis_ref, b_ref, o_ref):
    full = jnp.concatenate([acc_ref[0], acc_ref[1]], axis=1) + zs_ref[...]
    o_ref[...] = full * dis_ref[...] + b_ref[...]


def _tc_final(acc, zs, dis_col, b2):
    return pl.pallas_call(
        _final_body,
        grid=(N_NODES // ROWBLK,),
        in_specs=[
            pl.BlockSpec((NC, ROWBLK, DH), lambda i: (0, i, 0)),
            pl.BlockSpec((ROWBLK, D), lambda i: (i, 0)),
            pl.BlockSpec((ROWBLK, 1), lambda i: (i, 0)),
            pl.BlockSpec((1, D), lambda i: (0, 0)),
        ],
        out_specs=pl.BlockSpec((ROWBLK, D), lambda i: (i, 0)),
        out_shape=jax.ShapeDtypeStruct((N_NODES, D), F32),
    )(acc, zs, dis_col, b2)


def kernel(x, edge_index, weight, W1, b1, W2, b2):
    e = weight.shape[0]
    row = edge_index[0].astype(I32)
    col = edge_index[1].astype(I32)
    ew = weight.astype(F32)

    epw_d = ((e + NW * 16 - 1) // (NW * 16)) * 16
    e_pad_d = epw_d * NW
    col_d = jnp.pad(col, (0, e_pad_d - e))
    ew_d = jnp.pad(ew, (0, e_pad_d - e))

    nchunks = ((e + NS * K - 1) // (NS * K) + NBUF - 1) // NBUF * NBUF
    e_pad_m = NS * nchunks * K
    row_m = jnp.pad(row, (0, e_pad_m - e))
    col_m = jnp.pad(col, (0, e_pad_m - e))
    rc3 = (row_m | (col_m << 16)).reshape(NS, nchunks, K)
    ew3 = jnp.pad(ew, (0, e_pad_m - e)).reshape(NS, nchunks, K)

    z1 = _tc_matmul(x, W1)
    degp = _sc_degree(col_d, ew_d, epw_d)
    deg_col = jnp.sum(degp, axis=0).reshape(NPAD, 1)
    zs1, zsh1, dis_col = _tc_scale1(z1, deg_col)
    acc1 = _sc_message(zsh1, rc3, ew3, nchunks)
    zs2, zsh2 = _tc_mid(acc1, zs1, dis_col, W2, b1.reshape(1, D))
    acc2 = _sc_message(zsh2, rc3, ew3, nchunks)
    return _tc_final(acc2, zs2, dis_col, b2.reshape(1, D))


# speedup vs baseline: 1.9828x; 1.1238x over previous
import functools

import jax
import jax.numpy as jnp
from jax import lax
from jax.experimental import pallas as pl
from jax.experimental.pallas import tpu as pltpu
from jax.experimental.pallas import tpu_sc as plsc

N_NODES = 10000
NPAD = 10240
D = 128
NC = 2
NS = 16
NW = NC * NS
K = 64
NBUF = 3
ROWBLK = 1000
F32 = jnp.float32
I32 = jnp.int32


def _sc_mesh():
    return plsc.VectorSubcoreMesh(core_axis_name="c", subcore_axis_name="s")


def _deg_body(epw, col_hbm, ew_hbm, degp_hbm, colb, ewb, degv):
    cid = lax.axis_index("c")
    sid = lax.axis_index("s")
    wid = sid * NC + cid

    def zero(i, c):
        degv[pl.ds(i * 16, 16)] = jnp.zeros((16,), F32)
        return c

    lax.fori_loop(0, NPAD // 16, zero, 0)
    pltpu.sync_copy(col_hbm.at[pl.ds(wid * epw, epw)], colb)
    pltpu.sync_copy(ew_hbm.at[pl.ds(wid * epw, epw)], ewb)

    def edge(i, c):
        b = i * 16
        plsc.addupdate_scatter(degv, [colb[pl.ds(b, 16)]], ewb[pl.ds(b, 16)])
        return c

    lax.fori_loop(0, epw // 16, edge, 0)
    pltpu.sync_copy(degv, degp_hbm.at[wid])


def _sc_degree(col_p, ew_p, epw):
    kfn = functools.partial(
        pl.kernel,
        out_type=jax.ShapeDtypeStruct((NW, NPAD), F32),
        mesh=_sc_mesh(),
        compiler_params=pltpu.CompilerParams(needs_layout_passes=False),
        scratch_types=[
            pltpu.VMEM((epw,), I32),
            pltpu.VMEM((epw,), F32),
            pltpu.VMEM((NPAD,), F32),
        ],
    )(functools.partial(_deg_body, epw))
    return kfn(col_p, ew_p)


DH = D // NC
DQ = DH // 2


def _msg_body(nchunks, unroll, zsh_hbm, rc_hbm, ew_hbm, out_hbm,
              acc_sh, zsp, rcb, ewb, gb0, gb1, gb2, sb0, sb1,
              rv0, rv1, rv2, cv0, cv1,
              sg0, sg1, sg2, ss0, ss1):
    cid = lax.axis_index("c")
    sid = lax.axis_index("s")
    rows_per_tile = NPAD // NS
    gb = [gb0, gb1, gb2]
    sb = [sb0, sb1]
    rv = [rv0, rv1, rv2]
    cv = [cv0, cv1]
    sg = [sg0, sg1, sg2]
    ss = [ss0, ss1]
    himask = jnp.full((16,), -65536, I32)
    lomask = jnp.full((16,), 65535, I32)

    pltpu.sync_copy(rc_hbm.at[sid], rcb)
    pltpu.sync_copy(ew_hbm.at[sid], ewb)
    zrows = N_NODES // NS
    pltpu.sync_copy(zsh_hbm.at[cid, pl.ds(sid * zrows, zrows)],
                    zsp.at[pl.ds(sid * zrows, zrows)])

    def mat_rows(j, dst):
        rc_row = rcb.at[j]

        @plsc.parallel_loop(0, K // 16)
        def _(g):
            dst[pl.ds(g * 16, 16)] = rc_row[pl.ds(g * 16, 16)] & lomask

    def mat_cols(j, dst):
        rc_row = rcb.at[j]

        @plsc.parallel_loop(0, K // 16)
        def _(g):
            dst[pl.ds(g * 16, 16)] = lax.shift_right_logical(
                rc_row[pl.ds(g * 16, 16)], 16)

    @plsc.parallel_loop(0, K)
    def _(i):
        for g in range(DH // 16):
            sb0[i, pl.ds(g * 16, 16)] = jnp.zeros((16,), F32)

    def zacc(j, c):
        pltpu.sync_copy(sb0, acc_sh.at[pl.ds(sid * rows_per_tile + j * K, K)])
        return c

    lax.fori_loop(0, rows_per_tile // K, zacc, 0)

    plsc.subcore_barrier()

    for b in range(NBUF - 1):
        mat_rows(b, rv[b])
        pltpu.async_copy(zsp.at[rv[b]], gb[b], sg[b])

    def scale_chunk(gbuf, sbuf, j):
        ew_row = ewb.at[j]

        @plsc.parallel_loop(0, K, unroll=unroll)
        def _(k):
            spl = plsc.load_gather(ew_row, [jnp.full((16,), k, I32)])
            for g in range(DQ // 16):
                xi = gbuf[k, pl.ds(g * 16, 16)]
                lo = plsc.bitcast(lax.shift_left(xi, 16), F32)
                hi = plsc.bitcast(xi & himask, F32)
                sbuf[k, pl.ds(g * 16, 16)] = lo * spl
                sbuf[k, pl.ds(DQ + g * 16, 16)] = hi * spl

    def outer(jo, c):
        for s in range(6):
            j = jo * 6 + s
            b3 = s % 3
            bg3 = (s + 2) % 3
            b2 = s % 2
            pltpu.make_async_copy(zsp.at[rv[b3]], gb[b3], sg[b3]).wait()

            @pl.when(j + 2 < nchunks)
            def _():
                mat_rows(j + 2, rv[bg3])
                pltpu.async_copy(zsp.at[rv[bg3]], gb[bg3], sg[bg3])

            @pl.when(j >= 2)
            def _():
                pltpu.make_async_copy(sb[b2], acc_sh.at[cv[b2]], ss[b2]).wait()

            scale_chunk(gb[b3], sb[b2], j)
            mat_cols(j, cv[b2])
            pltpu.async_copy(sb[b2], acc_sh.at[cv[b2]], ss[b2], add=True)
        return c

    lax.fori_loop(0, nchunks // 6, outer, 0)

    for j in (nchunks - 2, nchunks - 1):
        b2 = j % 2
        pltpu.make_async_copy(sb[b2], acc_sh.at[cv[b2]], ss[b2]).wait()
    plsc.subcore_barrier()

    def wout(j, c):
        r = sid * rows_per_tile + j * K
        pltpu.sync_copy(acc_sh.at[pl.ds(r, K)], sb0)
        pltpu.sync_copy(sb0, out_hbm.at[cid, pl.ds(r, K)])
        return c

    lax.fori_loop(0, rows_per_tile // K, wout, 0)


def _sc_message(zsh, rc3, ew3, nchunks, unroll=4):
    kfn = functools.partial(
        pl.kernel,
        out_type=jax.ShapeDtypeStruct((NC, NPAD, DH), F32),
        mesh=_sc_mesh(),
        compiler_params=pltpu.CompilerParams(
            needs_layout_passes=False, use_tc_tiling_on_sc=False),
        scratch_types=[
            pltpu.VMEM_SHARED((NPAD, DH), F32),
            pltpu.VMEM_SHARED((N_NODES, DQ), I32),
            pltpu.VMEM((nchunks, K), I32),
            pltpu.VMEM((nchunks, K), F32),
        ] + [pltpu.VMEM((K, DQ), I32)] * 3
          + [pltpu.VMEM((K, DH), F32)] * 2
          + [pltpu.VMEM((K,), I32)] * 5
          + [pltpu.SemaphoreType.DMA] * 5,
    )(functools.partial(_msg_body, nchunks, unroll))
    return kfn(zsh, rc3, ew3)


def _mm_body(x_ref, w_ref, o_ref):
    o_ref[...] = jnp.dot(x_ref[...], w_ref[...], preferred_element_type=F32)


def _tc_matmul(x, w):
    return pl.pallas_call(
        _mm_body,
        grid=(N_NODES // ROWBLK,),
        in_specs=[
            pl.BlockSpec((ROWBLK, D), lambda i: (i, 0)),
            pl.BlockSpec((D, D), lambda i: (0, 0)),
        ],
        out_specs=pl.BlockSpec((ROWBLK, D), lambda i: (i, 0)),
        out_shape=jax.ShapeDtypeStruct((N_NODES, D), F32),
    )(x, w)


def _pack_halves(zs, oh_ref):
    for h in range(NC):
        a = zs[:, h * DH: h * DH + DQ]
        b = zs[:, h * DH + DQ: h * DH + 2 * DQ]
        ai = lax.bitcast_convert_type(a.astype(jnp.bfloat16).astype(F32), I32)
        bi = lax.bitcast_convert_type(b.astype(jnp.bfloat16).astype(F32), I32)
        oh_ref[h] = lax.shift_right_logical(ai, 16) | (bi & (-65536))


def _scale1_body(z_ref, deg_ref, zs_ref, zsh_ref, dis_ref):
    dis = lax.rsqrt(deg_ref[...] + 1.0)
    dis_ref[...] = dis
    zs = z_ref[...] * dis
    zs_ref[...] = zs
    _pack_halves(zs, zsh_ref)


def _tc_scale1(z, deg_col):
    return pl.pallas_call(
        _scale1_body,
        grid=(N_NODES // ROWBLK,),
        in_specs=[
            pl.BlockSpec((ROWBLK, D), lambda i: (i, 0)),
            pl.BlockSpec((ROWBLK, 1), lambda i: (i, 0)),
        ],
        out_specs=[
            pl.BlockSpec((ROWBLK, D), lambda i: (i, 0)),
            pl.BlockSpec((NC, ROWBLK, DQ), lambda i: (0, i, 0)),
            pl.BlockSpec((ROWBLK, 1), lambda i: (i, 0)),
        ],
        out_shape=[
            jax.ShapeDtypeStruct((N_NODES, D), F32),
            jax.ShapeDtypeStruct((NC, N_NODES, DQ), I32),
            jax.ShapeDtypeStruct((N_NODES, 1), F32),
        ],
    )(z, deg_col)


def _mid_body(acc_ref, zs_ref, dis_ref, w_ref, b_ref, o_ref, oh_ref):
    dis = dis_ref[...]
    full = jnp.concatenate([acc_ref[0], acc_ref[1]], axis=1) + zs_ref[...]
    pre = full * dis + b_ref[...]
    h = jnp.maximum(pre, 0.0)
    z2 = jnp.dot(h, w_ref[...], preferred_element_type=F32) * dis
    o_ref[...] = z2
    _pack_halves(z2, oh_ref)


def _tc_mid(acc, zs, dis_col, w2, b1):
    return pl.pallas_call(
        _mid_body,
        grid=(N_NODES // ROWBLK,),
        in_specs=[
            pl.BlockSpec((NC, ROWBLK, DH), lambda i: (0, i, 0)),
            pl.BlockSpec((ROWBLK, D), lambda i: (i, 0)),
            pl.BlockSpec((ROWBLK, 1), lambda i: (i, 0)),
            pl.BlockSpec((D, D), lambda i: (0, 0)),
            pl.BlockSpec((1, D), lambda i: (0, 0)),
        ],
        out_specs=[
            pl.BlockSpec((ROWBLK, D), lambda i: (i, 0)),
            pl.BlockSpec((NC, ROWBLK, DQ), lambda i: (0, i, 0)),
        ],
        out_shape=[
            jax.ShapeDtypeStruct((N_NODES, D), F32),
            jax.ShapeDtypeStruct((NC, N_NODES, DQ), I32),
        ],
    )(acc, zs, dis_col, w2, b1)


def _final_body(acc_ref, zs_ref, dis_ref, b_ref, o_ref):
    full = jnp.concatenate([acc_ref[0], acc_ref[1]], axis=1) + zs_ref[...]
    o_ref[...] = full * dis_ref[...] + b_ref[...]


def _tc_final(acc, zs, dis_col, b2):
    return pl.pallas_call(
        _final_body,
        grid=(N_NODES // ROWBLK,),
        in_specs=[
            pl.BlockSpec((NC, ROWBLK, DH), lambda i: (0, i, 0)),
            pl.BlockSpec((ROWBLK, D), lambda i: (i, 0)),
            pl.BlockSpec((ROWBLK, 1), lambda i: (i, 0)),
            pl.BlockSpec((1, D), lambda i: (0, 0)),
        ],
        out_specs=pl.BlockSpec((ROWBLK, D), lambda i: (i, 0)),
        out_shape=jax.ShapeDtypeStruct((N_NODES, D), F32),
    )(acc, zs, dis_col, b2)


def kernel(x, edge_index, weight, W1, b1, W2, b2):
    e = weight.shape[0]
    row = edge_index[0].astype(I32)
    col = edge_index[1].astype(I32)
    ew = weight.astype(F32)

    epw_d = ((e + NW * 16 - 1) // (NW * 16)) * 16
    e_pad_d = epw_d * NW
    col_d = jnp.pad(col, (0, e_pad_d - e))
    ew_d = jnp.pad(ew, (0, e_pad_d - e))

    nchunks = ((e + NS * K - 1) // (NS * K) + 5) // 6 * 6
    e_pad_m = NS * nchunks * K
    row_m = jnp.pad(row, (0, e_pad_m - e))
    col_m = jnp.pad(col, (0, e_pad_m - e))
    rc3 = (row_m | (col_m << 16)).reshape(NS, nchunks, K)
    ew3 = jnp.pad(ew, (0, e_pad_m - e)).reshape(NS, nchunks, K)

    z1 = _tc_matmul(x, W1)
    degp = _sc_degree(col_d, ew_d, epw_d)
    deg_col = jnp.sum(degp, axis=0).reshape(NPAD, 1)
    zs1, zsh1, dis_col = _tc_scale1(z1, deg_col)
    acc1 = _sc_message(zsh1, rc3, ew3, nchunks)
    zs2, zsh2 = _tc_mid(acc1, zs1, dis_col, W2, b1.reshape(1, D))
    acc2 = _sc_message(zsh2, rc3, ew3, nchunks)
    return _tc_final(acc2, zs2, dis_col, b2.reshape(1, D))
